# Initial kernel scaffold; baseline (speedup 1.0000x reference)
import jax
import jax.numpy as jnp
from jax.experimental import pallas as pl


def kernel(input, adj, weight, bias):
    raise NotImplementedError("write your pallas kernel here")


# speedup vs baseline: 11.2837x; 11.2837x over previous
import jax
import jax.numpy as jnp
from jax.experimental import pallas as pl
from jax.experimental.pallas import tpu as pltpu

N = 10000
F = 128
BC = 256
BR = 2000
NCB = (N + BC - 1) // BC
NPAD = NCB * BC
NRB = N // BR


def _support_body(x_ref, wh_ref, wl_ref, hi_ref, lo_ref):
    xb = x_ref[...]
    xh = xb.astype(jnp.bfloat16)
    xl = (xb - xh.astype(jnp.float32)).astype(jnp.bfloat16)
    s = jnp.dot(xh, wh_ref[...], preferred_element_type=jnp.float32)
    s += jnp.dot(xh, wl_ref[...], preferred_element_type=jnp.float32)
    s += jnp.dot(xl, wh_ref[...], preferred_element_type=jnp.float32)
    hi = s.astype(jnp.bfloat16)
    hi_ref[...] = hi
    lo_ref[...] = (s - hi.astype(jnp.float32)).astype(jnp.bfloat16)


def _pow2_neg(cnt_f32):
    e = jnp.maximum(127 - cnt_f32.astype(jnp.int32), 0) << 23
    return jax.lax.bitcast_convert_type(e, jnp.float32)


def _main_body(adj_ref, hi_ref, lo_ref, bias_ref, out_ref, scale_ref):
    j = pl.program_id(1)

    @pl.when(j == 0)
    def _init():
        scale_ref[...] = jnp.ones_like(scale_ref)
        out_ref[...] = jnp.zeros_like(out_ref)

    a = NCB - 1 - j
    adjb = adj_ref[...]
    ids = a * BC + jax.lax.broadcasted_iota(jnp.int32, (BR, BC), 1)
    m = jnp.where(ids < N, adjb, 0.0)

    ks = jax.lax.broadcasted_iota(jnp.int32, (BC, BC), 0)
    cs = jax.lax.broadcasted_iota(jnp.int32, (BC, BC), 1)
    tri = (ks > cs).astype(jnp.bfloat16)
    cnt = jnp.dot(m.astype(jnp.bfloat16), tri,
                  preferred_element_type=jnp.float32)

    w = (m * _pow2_neg(cnt)).astype(jnp.bfloat16)
    pc = jnp.dot(w, hi_ref[...], preferred_element_type=jnp.float32)
    pc += jnp.dot(w, lo_ref[...], preferred_element_type=jnp.float32)
    out_ref[...] += scale_ref[...] * pc

    tot = jnp.sum(m, axis=1, keepdims=True)
    scale_ref[...] *= _pow2_neg(tot)

    @pl.when(j == NCB - 1)
    def _fini():
        out_ref[...] += bias_ref[...]


def kernel(input, adj, weight, bias):
    x = jnp.pad(input, ((0, NPAD - N), (0, 0)))
    wh = weight.astype(jnp.bfloat16)
    wl = (weight - wh.astype(jnp.float32)).astype(jnp.bfloat16)

    hi, lo = pl.pallas_call(
        _support_body,
        grid=(NPAD // 1024,),
        in_specs=[
            pl.BlockSpec((1024, F), lambda i: (i, 0)),
            pl.BlockSpec((F, F), lambda i: (0, 0)),
            pl.BlockSpec((F, F), lambda i: (0, 0)),
        ],
        out_specs=[
            pl.BlockSpec((1024, F), lambda i: (i, 0)),
            pl.BlockSpec((1024, F), lambda i: (i, 0)),
        ],
        out_shape=[
            jax.ShapeDtypeStruct((NPAD, F), jnp.bfloat16),
            jax.ShapeDtypeStruct((NPAD, F), jnp.bfloat16),
        ],
    )(x, wh, wl)

    out = pl.pallas_call(
        _main_body,
        grid=(NRB, NCB),
        in_specs=[
            pl.BlockSpec((BR, BC), lambda i, j: (i, NCB - 1 - j)),
            pl.BlockSpec((BC, F), lambda i, j: (NCB - 1 - j, 0)),
            pl.BlockSpec((BC, F), lambda i, j: (NCB - 1 - j, 0)),
            pl.BlockSpec((1, F), lambda i, j: (0, 0)),
        ],
        out_specs=pl.BlockSpec((BR, F), lambda i, j: (i, 0)),
        out_shape=jax.ShapeDtypeStruct((N, F), jnp.float32),
        scratch_shapes=[pltpu.VMEM((BR, 1), jnp.float32)],
    )(adj, hi, lo, bias.reshape(1, F))
    return out


# speedup vs baseline: 12.4001x; 1.0989x over previous
import jax
import jax.numpy as jnp
import numpy as np
from jax.experimental import pallas as pl
from jax.experimental.pallas import tpu as pltpu

N = 10000
F = 128
BC = 256
BR = 2000
NCB = 40
NFULL = 39
TAIL = N - NFULL * BC
NPAD = NCB * BC
NRB = N // BR


def _support_body(x_ref, wh_ref, wl_ref, hi_ref, lo_ref):
    xb = x_ref[...]
    xh = xb.astype(jnp.bfloat16)
    xl = (xb - xh.astype(jnp.float32)).astype(jnp.bfloat16)
    s = jnp.dot(xh, wh_ref[...], preferred_element_type=jnp.float32)
    s += jnp.dot(xh, wl_ref[...], preferred_element_type=jnp.float32)
    s += jnp.dot(xl, wh_ref[...], preferred_element_type=jnp.float32)
    hi = s.astype(jnp.bfloat16)
    hi_ref[...] = hi
    lo_ref[...] = (s - hi.astype(jnp.float32)).astype(jnp.bfloat16)


def _main_body(adj_ref, tail_ref, tri_ref, ones_ref, hi_ref, lo_ref,
               bias_ref, out_ref, cum_ref):
    j = pl.program_id(1)
    hi = hi_ref[...].astype(jnp.float32)
    lo = lo_ref[...].astype(jnp.float32)

    @pl.when(j == 0)
    def _prologue():
        t = tail_ref[...]
        ks = jax.lax.broadcasted_iota(jnp.int32, (TAIL, TAIL), 0)
        cs = jax.lax.broadcasted_iota(jnp.int32, (TAIL, TAIL), 1)
        t16 = (ks > cs).astype(jnp.float32)
        cnt = jnp.dot(t, t16, preferred_element_type=jnp.float32)
        w = t * jnp.exp2(-cnt)
        out_ref[...] = (jnp.dot(w, hi[0:TAIL], preferred_element_type=jnp.float32)
                        + jnp.dot(w, lo[0:TAIL], preferred_element_type=jnp.float32))
        tot = jnp.sum(t, axis=1, keepdims=True)
        cum_ref[...] = jnp.broadcast_to(tot, (BR, 8))

    @pl.when(j > 0)
    def _block():
        m = adj_ref[...]
        m8 = m.astype(jnp.float8_e5m2)
        cnt = jnp.dot(m8, tri_ref[...], preferred_element_type=jnp.float32)
        p = jnp.exp2(-(cnt + cum_ref[:, 0:1]))
        w = m * p
        out_ref[...] += (jnp.dot(w, hi, preferred_element_type=jnp.float32)
                         + jnp.dot(w, lo, preferred_element_type=jnp.float32))
        tot = jnp.dot(m8, ones_ref[...], preferred_element_type=jnp.float32)
        cum_ref[...] += tot

    @pl.when(j == NCB - 1)
    def _fini():
        out_ref[...] += bias_ref[...]


def kernel(input, adj, weight, bias):
    x = jnp.pad(input, ((0, NPAD - N), (0, 0)))
    wh = weight.astype(jnp.bfloat16)
    wl = (weight - wh.astype(jnp.float32)).astype(jnp.bfloat16)

    hi, lo = pl.pallas_call(
        _support_body,
        grid=(NPAD // 1024,),
        in_specs=[
            pl.BlockSpec((1024, F), lambda i: (i, 0)),
            pl.BlockSpec((F, F), lambda i: (0, 0)),
            pl.BlockSpec((F, F), lambda i: (0, 0)),
        ],
        out_specs=[
            pl.BlockSpec((1024, F), lambda i: (i, 0)),
            pl.BlockSpec((1024, F), lambda i: (i, 0)),
        ],
        out_shape=[
            jax.ShapeDtypeStruct((NPAD, F), jnp.bfloat16),
            jax.ShapeDtypeStruct((NPAD, F), jnp.bfloat16),
        ],
    )(x, wh, wl)

    adj_tail = jax.lax.slice(adj, (0, NFULL * BC), (N, N))
    r = np.arange(BC)
    tri = jnp.asarray((r[:, None] > r[None, :]).astype(np.float32),
                      dtype=jnp.float8_e5m2)
    ones8 = jnp.ones((BC, 8), dtype=jnp.float8_e5m2)

    out = pl.pallas_call(
        _main_body,
        grid=(NRB, NCB),
        in_specs=[
            pl.BlockSpec((BR, BC),
                         lambda i, j: (i, NFULL - jnp.maximum(j, 1))),
            pl.BlockSpec((BR, TAIL), lambda i, j: (i, 0)),
            pl.BlockSpec((BC, BC), lambda i, j: (0, 0)),
            pl.BlockSpec((BC, 8), lambda i, j: (0, 0)),
            pl.BlockSpec((BC, F), lambda i, j: (NCB - 1 - j, 0)),
            pl.BlockSpec((BC, F), lambda i, j: (NCB - 1 - j, 0)),
            pl.BlockSpec((1, F), lambda i, j: (0, 0)),
        ],
        out_specs=pl.BlockSpec((BR, F), lambda i, j: (i, 0)),
        out_shape=jax.ShapeDtypeStruct((N, F), jnp.float32),
        scratch_shapes=[pltpu.VMEM((BR, 8), jnp.float32)],
    )(adj, adj_tail, tri, ones8, hi, lo, bias.reshape(1, F))
    return out


# speedup vs baseline: 15.0389x; 1.2128x over previous
import jax
import jax.numpy as jnp
import numpy as np
from jax.experimental import pallas as pl
from jax.experimental.pallas import tpu as pltpu

N = 10000
F = 128
BC = 256
CPB = 3
BD = BC * CPB
BR = 2000
NFULL = 13
NCB = NFULL + 1
TAIL = N - NFULL * BD
NPAD = NCB * BD
NRB = N // BR


def _support_body(x_ref, wh_ref, wl_ref, hi_ref, lo_ref):
    xb = x_ref[...]
    xh = xb.astype(jnp.bfloat16)
    xl = (xb - xh.astype(jnp.float32)).astype(jnp.bfloat16)
    s = jnp.dot(xh, wh_ref[...], preferred_element_type=jnp.float32)
    s += jnp.dot(xh, wl_ref[...], preferred_element_type=jnp.float32)
    s += jnp.dot(xl, wh_ref[...], preferred_element_type=jnp.float32)
    hi = s.astype(jnp.bfloat16)
    hi_ref[...] = hi
    lo_ref[...] = (s - hi.astype(jnp.float32)).astype(jnp.bfloat16)


def _main_body(adj_ref, tail_ref, tri_ref, ones_ref, hi_ref, lo_ref,
               bias_ref, out_ref, cum_ref):
    j = pl.program_id(1)

    @pl.when(j == 0)
    def _prologue():
        t = tail_ref[...]
        ks = jax.lax.broadcasted_iota(jnp.int32, (TAIL, TAIL), 0)
        cs = jax.lax.broadcasted_iota(jnp.int32, (TAIL, TAIL), 1)
        t16 = (ks > cs).astype(jnp.float32)
        cnt = jnp.dot(t, t16, preferred_element_type=jnp.float32)
        w = (t * jnp.exp2(-cnt)).astype(jnp.bfloat16)
        out_ref[...] = (
            jnp.dot(w, hi_ref[0:TAIL], preferred_element_type=jnp.float32)
            + jnp.dot(w, lo_ref[0:TAIL], preferred_element_type=jnp.float32))
        tot = jnp.sum(t, axis=1, keepdims=True)
        cum_ref[...] = jnp.broadcast_to(tot, (BR, 8))

    @pl.when(j > 0)
    def _block():
        tri = tri_ref[...]
        ones8 = ones_ref[...]
        c_loc = cum_ref[:, 0:1]
        acc = jnp.zeros((BR, F), dtype=jnp.float32)
        tots = jnp.zeros((BR, 8), dtype=jnp.float32)
        for k in range(CPB - 1, -1, -1):
            m = adj_ref[:, k * BC:(k + 1) * BC]
            m8 = m.astype(jnp.float8_e5m2)
            cnt = jnp.dot(m8, tri, preferred_element_type=jnp.float32)
            p = jnp.exp2(-(cnt + c_loc))
            w = (m * p).astype(jnp.bfloat16)
            hi = hi_ref[k * BC:(k + 1) * BC, :]
            lo = lo_ref[k * BC:(k + 1) * BC, :]
            acc += jnp.dot(w, hi, preferred_element_type=jnp.float32)
            acc += jnp.dot(w, lo, preferred_element_type=jnp.float32)
            tot = jnp.dot(m8, ones8, preferred_element_type=jnp.float32)
            tots += tot
            c_loc = c_loc + tot[:, 0:1]
        out_ref[...] += acc
        cum_ref[...] += tots

    @pl.when(j == NCB - 1)
    def _fini():
        out_ref[...] += bias_ref[...]


def kernel(input, adj, weight, bias):
    x = jnp.pad(input, ((0, NPAD - N), (0, 0)))
    wh = weight.astype(jnp.bfloat16)
    wl = (weight - wh.astype(jnp.float32)).astype(jnp.bfloat16)

    hi, lo = pl.pallas_call(
        _support_body,
        grid=(NPAD // BD,),
        in_specs=[
            pl.BlockSpec((BD, F), lambda i: (i, 0)),
            pl.BlockSpec((F, F), lambda i: (0, 0)),
            pl.BlockSpec((F, F), lambda i: (0, 0)),
        ],
        out_specs=[
            pl.BlockSpec((BD, F), lambda i: (i, 0)),
            pl.BlockSpec((BD, F), lambda i: (i, 0)),
        ],
        out_shape=[
            jax.ShapeDtypeStruct((NPAD, F), jnp.bfloat16),
            jax.ShapeDtypeStruct((NPAD, F), jnp.bfloat16),
        ],
    )(x, wh, wl)

    adj_tail = jax.lax.slice(adj, (0, NFULL * BD), (N, N))
    r = np.arange(BC)
    tri = jnp.asarray((r[:, None] > r[None, :]).astype(np.float32),
                      dtype=jnp.float8_e5m2)
    ones8 = jnp.ones((BC, 8), dtype=jnp.float8_e5m2)

    out = pl.pallas_call(
        _main_body,
        grid=(NRB, NCB),
        in_specs=[
            pl.BlockSpec((BR, BD),
                         lambda i, j: (i, NFULL - jnp.maximum(j, 1))),
            pl.BlockSpec((BR, TAIL), lambda i, j: (i, 0)),
            pl.BlockSpec((BC, BC), lambda i, j: (0, 0)),
            pl.BlockSpec((BC, 8), lambda i, j: (0, 0)),
            pl.BlockSpec((BD, F), lambda i, j: (NCB - 1 - j, 0)),
            pl.BlockSpec((BD, F), lambda i, j: (NCB - 1 - j, 0)),
            pl.BlockSpec((1, F), lambda i, j: (0, 0)),
        ],
        out_specs=pl.BlockSpec((BR, F), lambda i, j: (i, 0)),
        out_shape=jax.ShapeDtypeStruct((N, F), jnp.float32),
        scratch_shapes=[pltpu.VMEM((BR, 8), jnp.float32)],
    )(adj, adj_tail, tri, ones8, hi, lo, bias.reshape(1, F))
    return out


# speedup vs baseline: 19.0711x; 1.2681x over previous
import jax
import jax.numpy as jnp
import numpy as np
from jax.experimental import pallas as pl
from jax.experimental.pallas import tpu as pltpu

N = 10000
F = 128
BC = 256
CPB = 3
BD = BC * CPB
BR = 2000
NFULL = 13
NCB = NFULL + 1
TAIL = N - NFULL * BD
NPAD = NCB * BD
NRB = N // BR


def _support_body(x_ref, wh_ref, wl_ref, hi_ref):
    xb = x_ref[...]
    xh = xb.astype(jnp.bfloat16)
    xl = (xb - xh.astype(jnp.float32)).astype(jnp.bfloat16)
    s = jnp.dot(xh, wh_ref[...], preferred_element_type=jnp.float32)
    s += jnp.dot(xh, wl_ref[...], preferred_element_type=jnp.float32)
    s += jnp.dot(xl, wh_ref[...], preferred_element_type=jnp.float32)
    hi_ref[...] = s.astype(jnp.bfloat16)


def _main_body(adj_ref, tail_ref, tri_ref, hi_ref, bias_ref, out_ref, cum_ref):
    j = pl.program_id(1)

    @pl.when(j == 0)
    def _prologue():
        t = tail_ref[...]
        ks = jax.lax.broadcasted_iota(jnp.int32, (TAIL, TAIL), 0)
        cs = jax.lax.broadcasted_iota(jnp.int32, (TAIL, TAIL), 1)
        t16 = (ks > cs).astype(jnp.float32)
        cnt = jnp.dot(t, t16, preferred_element_type=jnp.float32)
        w = (t * jnp.exp2(-cnt)).astype(jnp.bfloat16)
        out_ref[...] = jnp.dot(w, hi_ref[NFULL * BD:NFULL * BD + TAIL, :],
                               preferred_element_type=jnp.float32)
        cum_ref[...] = jnp.sum(t, axis=1, keepdims=True)

    @pl.when(j > 0)
    def _block():
        a = NFULL - j
        tri = tri_ref[...]
        mt = adj_ref[...]
        m8 = mt.astype(jnp.float8_e5m2)
        acc = jnp.zeros((BR, F), dtype=jnp.float32)
        neg = 1.0 - cum_ref[...]
        for k in range(CPB - 1, -1, -1):
            cnt = jnp.dot(m8[:, k * BC:(k + 1) * BC], tri,
                          preferred_element_type=jnp.float32)
            p = neg.astype(jnp.bfloat16) - cnt.astype(jnp.bfloat16)
            w = mt[:, k * BC:(k + 1) * BC].astype(jnp.bfloat16) * jnp.exp2(p)
            hi = hi_ref[pl.ds(a * BD + k * BC, BC), :]
            acc += jnp.dot(w, hi, preferred_element_type=jnp.float32)
            neg = neg - cnt[:, 0:1]
        out_ref[...] += acc
        cum_ref[...] = 1.0 - neg

    @pl.when(j == NCB - 1)
    def _fini():
        out_ref[...] += bias_ref[...]


def kernel(input, adj, weight, bias):
    x = jnp.pad(input, ((0, NPAD - N), (0, 0)))
    wh = weight.astype(jnp.bfloat16)
    wl = (weight - wh.astype(jnp.float32)).astype(jnp.bfloat16)

    hi = pl.pallas_call(
        _support_body,
        grid=(NPAD // BD,),
        in_specs=[
            pl.BlockSpec((BD, F), lambda i: (i, 0)),
            pl.BlockSpec((F, F), lambda i: (0, 0)),
            pl.BlockSpec((F, F), lambda i: (0, 0)),
        ],
        out_specs=pl.BlockSpec((BD, F), lambda i: (i, 0)),
        out_shape=jax.ShapeDtypeStruct((NPAD, F), jnp.bfloat16),
    )(x, wh, wl)

    adj_tail = jax.lax.slice(adj, (0, NFULL * BD), (N, N))
    r = np.arange(BC)
    tri = jnp.asarray((r[:, None] >= r[None, :]).astype(np.float32),
                      dtype=jnp.float8_e5m2)

    out = pl.pallas_call(
        _main_body,
        grid=(NRB, NCB),
        in_specs=[
            pl.BlockSpec((BR, BD),
                         lambda i, j: (i, NFULL - jnp.maximum(j, 1))),
            pl.BlockSpec((BR, TAIL), lambda i, j: (i, 0)),
            pl.BlockSpec((BC, BC), lambda i, j: (0, 0)),
            pl.BlockSpec((NPAD, F), lambda i, j: (0, 0)),
            pl.BlockSpec((1, F), lambda i, j: (0, 0)),
        ],
        out_specs=pl.BlockSpec((BR, F), lambda i, j: (i, 0)),
        out_shape=jax.ShapeDtypeStruct((N, F), jnp.float32),
        scratch_shapes=[pltpu.VMEM((BR, 1), jnp.float32)],
    )(adj, adj_tail, tri, hi, bias.reshape(1, F))
    return out
